# Initial kernel scaffold; baseline (speedup 1.0000x reference)
import jax
import jax.numpy as jnp
from jax.experimental import pallas as pl


def kernel(source_x, source_edge_index, source_batch, target_x, target_edge_index, target_batch, Wl1, bl1, Wr1, br1, att1, bias1, Wl2, bl2, Wr2, br2, att2, bias2, pool_w1, pool_w2, W1, b1, W2, b2, W3, b3):
    raise NotImplementedError("write your pallas kernel here")


# speedup vs baseline: 1.0275x; 1.0275x over previous
import functools

import jax
import jax.numpy as jnp
from jax.experimental import pallas as pl
from jax.experimental.pallas import tpu as pltpu

N = 10000
E = 320000
G = 16
RATIO = 0.5


def _lin2_body(x_ref, wl_ref, bl_ref, wr_ref, br_ref, xl_ref, xr_ref):
    x = x_ref[...]
    xl_ref[...] = x @ wl_ref[...] + bl_ref[...][None, :]
    xr_ref[...] = x @ wr_ref[...] + br_ref[...][None, :]


def _linear2(x, Wl, bl, Wr, br):
    n, d = x.shape
    h = Wl.shape[1]
    blk = 2000
    grid = n // blk
    return pl.pallas_call(
        _lin2_body,
        grid=(grid,),
        in_specs=[
            pl.BlockSpec((blk, d), lambda i: (i, 0)),
            pl.BlockSpec((d, h), lambda i: (0, 0)),
            pl.BlockSpec((h,), lambda i: (0,)),
            pl.BlockSpec((d, h), lambda i: (0, 0)),
            pl.BlockSpec((h,), lambda i: (0,)),
        ],
        out_specs=[
            pl.BlockSpec((blk, h), lambda i: (i, 0)),
            pl.BlockSpec((blk, h), lambda i: (i, 0)),
        ],
        out_shape=[
            jax.ShapeDtypeStruct((n, h), jnp.float32),
            jax.ShapeDtypeStruct((n, h), jnp.float32),
        ],
    )(x, Wl, bl, Wr, br)


def _gatv2(x, edge_index, Wl, bl, Wr, br, att, bias):
    n = x.shape[0]
    loops = jnp.arange(n, dtype=edge_index.dtype)
    src = jnp.concatenate([edge_index[0], loops])
    dst = jnp.concatenate([edge_index[1], loops])
    xl, xr = _linear2(x, Wl, bl, Wr, br)
    e = jax.nn.leaky_relu(xl[src] + xr[dst], negative_slope=0.2) @ att
    m = jax.ops.segment_max(e, dst, num_segments=n)
    m = jnp.where(jnp.isfinite(m), m, 0.0)
    ex = jnp.exp(e - m[dst])
    denom = jax.ops.segment_sum(ex, dst, num_segments=n)
    alpha = ex / denom[dst]
    out = jax.ops.segment_sum(alpha[:, None] * xl[src], dst, num_segments=n)
    return out + bias


def _topk_pool(x, batch, weight):
    n = x.shape[0]
    score = jnp.tanh((x @ weight) / jnp.linalg.norm(weight))
    order = jnp.argsort(batch.astype(jnp.float32) * 4.0 - score)
    counts = jnp.bincount(batch, length=G)
    offsets = jnp.concatenate([jnp.zeros((1,), counts.dtype), jnp.cumsum(counts)[:-1]])
    batch_sorted = batch[order]
    rank_sorted = jnp.arange(n) - offsets[batch_sorted]
    rank = jnp.zeros((n,), jnp.int32).at[order].set(rank_sorted.astype(jnp.int32))
    k = jnp.ceil(RATIO * counts.astype(jnp.float32)).astype(jnp.int32)
    keep = rank < k[batch]
    return x * score[:, None], keep


def _global_pools(x, batch, keep):
    mask = keep[:, None].astype(x.dtype)
    neg = jnp.where(keep[:, None], x, -jnp.inf)
    gmax = jax.ops.segment_max(neg, batch, num_segments=G)
    gmax = jnp.where(jnp.isfinite(gmax), gmax, 0.0)
    s = jax.ops.segment_sum(x * mask, batch, num_segments=G)
    cnt = jax.ops.segment_sum(mask, batch, num_segments=G)
    gmean = s / jnp.maximum(cnt, 1.0)
    return jnp.concatenate([gmax, gmean], axis=1)


def kernel(source_x, source_edge_index, source_batch, target_x, target_edge_index, target_batch, Wl1, bl1, Wr1, br1, att1, bias1, Wl2, bl2, Wr2, br2, att2, bias2, pool_w1, pool_w2, W1, b1, W2, b2, W3, b3):
    s = jax.nn.leaky_relu(_gatv2(source_x, source_edge_index, Wl1, bl1, Wr1, br1, att1, bias1), negative_slope=0.01)
    s, keep_s = _topk_pool(s, source_batch, pool_w1)
    s_feat = _global_pools(s, source_batch, keep_s)
    t = jax.nn.leaky_relu(_gatv2(target_x, target_edge_index, Wl2, bl2, Wr2, br2, att2, bias2), negative_slope=0.01)
    t, keep_t = _topk_pool(t, target_batch, pool_w2)
    t_feat = _global_pools(t, target_batch, keep_t)
    h = jnp.concatenate([s_feat, t_feat], axis=1)
    h = jax.nn.leaky_relu(h @ W1 + b1, negative_slope=0.01)
    h = jax.nn.leaky_relu(h @ W2 + b2, negative_slope=0.01)
    h = h @ W3 + b3
    return jax.nn.log_softmax(h, axis=-1)


# speedup vs baseline: 1.0958x; 1.0665x over previous
import functools

import jax
import jax.numpy as jnp
from jax import lax
from jax.experimental import pallas as pl
from jax.experimental.pallas import tpu as pltpu
from jax.experimental.pallas import tpu_sc as plsc

N = 10000
E = 320000
G = 16
RATIO = 0.5

NC = 2
NS = 16
NW = NC * NS
L = 16

EP = E + N
CHUNK = 128
TPW = 10368
NCHUNK = TPW // CHUNK
EPAD = NW * TPW

_SC_MESH = plsc.VectorSubcoreMesh(
    core_axis_name="c", subcore_axis_name="s", num_cores=NC, num_subcores=NS
)

NEG = -1e30


def _lin2_body(x_ref, wl_ref, bl_ref, wr_ref, br_ref, xl_ref, xr_ref):
    x = x_ref[...]
    xl_ref[...] = x @ wl_ref[...] + bl_ref[...][None, :]
    xr_ref[...] = x @ wr_ref[...] + br_ref[...][None, :]


def _linear2(x, Wl, bl, Wr, br):
    n, d = x.shape
    h = Wl.shape[1]
    blk = 2000
    grid = n // blk
    return pl.pallas_call(
        _lin2_body,
        grid=(grid,),
        in_specs=[
            pl.BlockSpec((blk, d), lambda i: (i, 0)),
            pl.BlockSpec((d, h), lambda i: (0, 0)),
            pl.BlockSpec((h,), lambda i: (0,)),
            pl.BlockSpec((d, h), lambda i: (0, 0)),
            pl.BlockSpec((h,), lambda i: (0,)),
        ],
        out_specs=[
            pl.BlockSpec((blk, h), lambda i: (i, 0)),
            pl.BlockSpec((blk, h), lambda i: (i, 0)),
        ],
        out_shape=[
            jax.ShapeDtypeStruct((n, h), jnp.float32),
            jax.ShapeDtypeStruct((n, h), jnp.float32),
        ],
    )(x, Wl, bl, Wr, br)


def _edge_score_body(xl_hbm, xr_hbm, src_hbm, dst_hbm, att_hbm, e_hbm,
                     src_v, dst_v, att_v, xl_rows, xr_rows, e_all, sem_a, sem_b):
    wid = lax.axis_index("s") * NC + lax.axis_index("c")
    pltpu.sync_copy(src_hbm.at[wid], src_v)
    pltpu.sync_copy(dst_hbm.at[wid], dst_v)
    pltpu.sync_copy(att_hbm, att_v)

    def chunk_body(ci, _):
        ca = pltpu.async_copy(xl_hbm.at[src_v.at[ci]], xl_rows, sem_a)
        cb = pltpu.async_copy(xr_hbm.at[dst_v.at[ci]], xr_rows, sem_b)
        ca.wait()
        cb.wait()

        lane = lax.iota(jnp.int32, L)
        gid0 = (wid * NCHUNK + ci) * CHUNK
        att_vecs = [att_v[pl.ds(t * L, L)] for t in range(8)]

        def group_body(g, _):
            evec = jnp.zeros((L,), jnp.float32)
            for k in range(L):
                acc = jnp.zeros((L,), jnp.float32)
                for j in range(8):
                    a = xl_rows[g * L + k, pl.ds(j * L, L)]
                    b = xr_rows[g * L + k, pl.ds(j * L, L)]
                    z = a + b
                    acc = acc + att_vecs[j] * jnp.maximum(z, z * 0.2)
                evec = jnp.where(lane == k, jnp.sum(acc), evec)
            gid = gid0 + g * L + lane
            e_all[ci, pl.ds(g * L, L)] = jnp.where(gid < EP, evec, NEG)
            return 0

        lax.fori_loop(0, CHUNK // L, group_body, 0)
        return 0

    lax.fori_loop(0, NCHUNK, chunk_body, 0)
    pltpu.sync_copy(e_all, e_hbm.at[wid])


@functools.partial(jax.jit, static_argnames=())
def _edge_scores(xl, xr, src2, dst2, att):
    f = pl.kernel(
        _edge_score_body,
        out_type=jax.ShapeDtypeStruct((NW, NCHUNK, CHUNK), jnp.float32),
        mesh=_SC_MESH,
        compiler_params=pltpu.CompilerParams(needs_layout_passes=False),
        scratch_types=[
            pltpu.VMEM((NCHUNK, CHUNK), jnp.int32),
            pltpu.VMEM((NCHUNK, CHUNK), jnp.int32),
            pltpu.VMEM((128,), jnp.float32),
            pltpu.VMEM((CHUNK, 128), jnp.float32),
            pltpu.VMEM((CHUNK, 128), jnp.float32),
            pltpu.VMEM((NCHUNK, CHUNK), jnp.float32),
            pltpu.SemaphoreType.DMA,
            pltpu.SemaphoreType.DMA,
        ],
    )
    return f(xl, xr, src2, dst2, att)


def _gatv2(x, edge_index, Wl, bl, Wr, br, att, bias):
    n = x.shape[0]
    loops = jnp.arange(n, dtype=edge_index.dtype)
    pad = jnp.zeros((EPAD - EP,), edge_index.dtype)
    src = jnp.concatenate([edge_index[0], loops, pad])
    dst = jnp.concatenate([edge_index[1], loops, pad])
    src2 = src.reshape(NW, NCHUNK, CHUNK)
    dst2 = dst.reshape(NW, NCHUNK, CHUNK)
    xl, xr = _linear2(x, Wl, bl, Wr, br)
    e = _edge_scores(xl, xr, src2, dst2, att).reshape(EPAD)[:EP]
    dst_t = dst[:EP]
    src_t = src[:EP]
    m = jax.ops.segment_max(e, dst_t, num_segments=n)
    m = jnp.where(jnp.isfinite(m), m, 0.0)
    ex = jnp.exp(e - m[dst_t])
    denom = jax.ops.segment_sum(ex, dst_t, num_segments=n)
    alpha = ex / denom[dst_t]
    out = jax.ops.segment_sum(alpha[:, None] * xl[src_t], dst_t, num_segments=n)
    return out + bias


def _topk_pool(x, batch, weight):
    n = x.shape[0]
    score = jnp.tanh((x @ weight) / jnp.linalg.norm(weight))
    order = jnp.argsort(batch.astype(jnp.float32) * 4.0 - score)
    counts = jnp.bincount(batch, length=G)
    offsets = jnp.concatenate([jnp.zeros((1,), counts.dtype), jnp.cumsum(counts)[:-1]])
    batch_sorted = batch[order]
    rank_sorted = jnp.arange(n) - offsets[batch_sorted]
    rank = jnp.zeros((n,), jnp.int32).at[order].set(rank_sorted.astype(jnp.int32))
    k = jnp.ceil(RATIO * counts.astype(jnp.float32)).astype(jnp.int32)
    keep = rank < k[batch]
    return x * score[:, None], keep


def _global_pools(x, batch, keep):
    mask = keep[:, None].astype(x.dtype)
    neg = jnp.where(keep[:, None], x, -jnp.inf)
    gmax = jax.ops.segment_max(neg, batch, num_segments=G)
    gmax = jnp.where(jnp.isfinite(gmax), gmax, 0.0)
    s = jax.ops.segment_sum(x * mask, batch, num_segments=G)
    cnt = jax.ops.segment_sum(mask, batch, num_segments=G)
    gmean = s / jnp.maximum(cnt, 1.0)
    return jnp.concatenate([gmax, gmean], axis=1)


def kernel(source_x, source_edge_index, source_batch, target_x, target_edge_index, target_batch, Wl1, bl1, Wr1, br1, att1, bias1, Wl2, bl2, Wr2, br2, att2, bias2, pool_w1, pool_w2, W1, b1, W2, b2, W3, b3):
    s = jax.nn.leaky_relu(_gatv2(source_x, source_edge_index, Wl1, bl1, Wr1, br1, att1, bias1), negative_slope=0.01)
    s, keep_s = _topk_pool(s, source_batch, pool_w1)
    s_feat = _global_pools(s, source_batch, keep_s)
    t = jax.nn.leaky_relu(_gatv2(target_x, target_edge_index, Wl2, bl2, Wr2, br2, att2, bias2), negative_slope=0.01)
    t, keep_t = _topk_pool(t, target_batch, pool_w2)
    t_feat = _global_pools(t, target_batch, keep_t)
    h = jnp.concatenate([s_feat, t_feat], axis=1)
    h = jax.nn.leaky_relu(h @ W1 + b1, negative_slope=0.01)
    h = jax.nn.leaky_relu(h @ W2 + b2, negative_slope=0.01)
    h = h @ W3 + b3
    return jax.nn.log_softmax(h, axis=-1)


# speedup vs baseline: 5.8484x; 5.3369x over previous
import jax
import jax.numpy as jnp
from jax import lax
from jax.experimental import pallas as pl
from jax.experimental.pallas import tpu as pltpu
from jax.experimental.pallas import tpu_sc as plsc

N = 10000
E = 320000
G = 16
RATIO = 0.5

NC = 2
NS = 16
NW = NC * NS
L = 16

EP = E + N
CHUNK = 128
TPW = 10368
NCHUNK = TPW // CHUNK
EPAD = NW * TPW

NPAD = 10240
W = NPAD // NS

NEG = -1e30

_SC_MESH = plsc.VectorSubcoreMesh(
    core_axis_name="c", subcore_axis_name="s", num_cores=NC, num_subcores=NS
)
_SC_PARAMS = pltpu.CompilerParams(needs_layout_passes=False)


def _lin2_body(x_ref, wl_ref, bl_ref, wr_ref, br_ref, xl_ref, xr_ref):
    x = x_ref[...]
    xl_ref[...] = x @ wl_ref[...] + bl_ref[...][None, :]
    xr_ref[...] = x @ wr_ref[...] + br_ref[...][None, :]


def _linear2(x, Wl, bl, Wr, br):
    n, d = x.shape
    h = Wl.shape[1]
    blk = 2000
    return pl.pallas_call(
        _lin2_body,
        grid=(n // blk,),
        in_specs=[
            pl.BlockSpec((blk, d), lambda i: (i, 0)),
            pl.BlockSpec((d, h), lambda i: (0, 0)),
            pl.BlockSpec((h,), lambda i: (0,)),
            pl.BlockSpec((d, h), lambda i: (0, 0)),
            pl.BlockSpec((h,), lambda i: (0,)),
        ],
        out_specs=[
            pl.BlockSpec((blk, h), lambda i: (i, 0)),
            pl.BlockSpec((blk, h), lambda i: (i, 0)),
        ],
        out_shape=[
            jax.ShapeDtypeStruct((n, h), jnp.float32),
            jax.ShapeDtypeStruct((n, h), jnp.float32),
        ],
    )(x, Wl, bl, Wr, br)


def _lane_gather(v, idx):
    dnums = lax.GatherDimensionNumbers(
        offset_dims=(), collapsed_slice_dims=(0,), start_index_map=(0,))
    return lax.gather(v, idx[:, None], dnums, (1,),
                      mode=lax.GatherScatterMode.PROMISE_IN_BOUNDS)


def _run_scan(lane, ds_, v, op):
    for s in (1, 2, 4, 8):
        idx = jnp.maximum(lane - s, 0)
        vs = _lane_gather(v, idx)
        ks = _lane_gather(ds_, idx)
        v = jnp.where((ks == ds_) & (lane >= s), op(v, vs), v)
    return v


def _run_end(lane, ds_):
    nidx = jnp.minimum(lane + 1, L - 1)
    dn = _lane_gather(ds_, nidx)
    return (ds_ != dn) | (lane == L - 1)


def _edge_score_body(xl0_hbm, xr0_hbm, xl1_hbm, xr1_hbm, src_hbm, dst_hbm,
                     att_hbm, e_hbm,
                     src_v, dst_v, att_v, xl_rows, xr_rows, e_all, sem_a, sem_b):
    wid = lax.axis_index("s") * NC + lax.axis_index("c")
    lane = lax.iota(jnp.int32, L)
    for gi, (xl_hbm, xr_hbm) in enumerate(((xl0_hbm, xr0_hbm),
                                           (xl1_hbm, xr1_hbm))):
        pltpu.sync_copy(src_hbm.at[gi, wid], src_v)
        pltpu.sync_copy(dst_hbm.at[gi, wid], dst_v)
        pltpu.sync_copy(att_hbm.at[gi], att_v)
        att_vecs = [att_v[pl.ds(t * L, L)] for t in range(8)]

        def chunk_body(ci, _):
            ca = pltpu.async_copy(xl_hbm.at[src_v.at[ci]], xl_rows, sem_a)
            cb = pltpu.async_copy(xr_hbm.at[dst_v.at[ci]], xr_rows, sem_b)
            ca.wait()
            cb.wait()
            gid0 = (wid * NCHUNK + ci) * CHUNK

            def group_body(g, _):
                evec = jnp.zeros((L,), jnp.float32)
                for k in range(L):
                    acc = jnp.zeros((L,), jnp.float32)
                    for j in range(8):
                        a = xl_rows[g * L + k, pl.ds(j * L, L)]
                        b = xr_rows[g * L + k, pl.ds(j * L, L)]
                        z = a + b
                        acc = acc + att_vecs[j] * jnp.maximum(z, z * 0.2)
                    evec = jnp.where(lane == k, jnp.sum(acc), evec)
                gid = gid0 + g * L + lane
                e_all[ci, pl.ds(g * L, L)] = jnp.where(gid < EP, evec, NEG)
                return 0

            lax.fori_loop(0, CHUNK // L, group_body, 0)
            return 0

        lax.fori_loop(0, NCHUNK, chunk_body, 0)
        pltpu.sync_copy(e_all, e_hbm.at[gi, wid])


def _edge_scores(xl0, xr0, xl1, xr1, src2, dst2, att2g):
    f = pl.kernel(
        _edge_score_body,
        out_type=jax.ShapeDtypeStruct((2, NW, NCHUNK, CHUNK), jnp.float32),
        mesh=_SC_MESH,
        compiler_params=_SC_PARAMS,
        scratch_types=[
            pltpu.VMEM((NCHUNK, CHUNK), jnp.int32),
            pltpu.VMEM((NCHUNK, CHUNK), jnp.int32),
            pltpu.VMEM((128,), jnp.float32),
            pltpu.VMEM((CHUNK, 128), jnp.float32),
            pltpu.VMEM((CHUNK, 128), jnp.float32),
            pltpu.VMEM((NCHUNK, CHUNK), jnp.float32),
            pltpu.SemaphoreType.DMA,
            pltpu.SemaphoreType.DMA,
        ],
    )
    return f(xl0, xr0, xl1, xr1, src2, dst2, att2g)


def _seg_max_body(dst_hbm, e_hbm, m_hbm, d_v, e_v, m_tile, shared, acc_v, tmp_v):
    cid = lax.axis_index("c")
    sid = lax.axis_index("s")
    wid = sid * NC + cid
    lane = lax.iota(jnp.int32, L)
    col0 = sid * W
    for gi in range(2):
        pltpu.sync_copy(dst_hbm.at[gi, wid], d_v)
        pltpu.sync_copy(e_hbm.at[gi, wid], e_v)

        def init_body(i, _):
            m_tile[pl.ds(i * L, L)] = jnp.full((L,), NEG, jnp.float32)
            return 0

        lax.fori_loop(0, NPAD // L, init_body, 0)

        def chunk_body(ci, _):
            def group_body(g, _):
                d = d_v[ci, pl.ds(g * L, L)]
                e = e_v[ci, pl.ds(g * L, L)]
                ds_, es_ = lax.sort((d, e), num_keys=1)
                v = _run_scan(lane, ds_, es_, jnp.maximum)
                mask = _run_end(lane, ds_)
                cur = plsc.load_gather(m_tile, [ds_])
                plsc.store_scatter(m_tile, [ds_], jnp.maximum(cur, v), mask=mask)
                return 0

            lax.fori_loop(0, CHUNK // L, group_body, 0)
            return 0

        lax.fori_loop(0, NCHUNK, chunk_body, 0)

        pltpu.sync_copy(m_tile, shared.at[sid])
        plsc.subcore_barrier()
        pltpu.sync_copy(shared.at[pl.ds(0, 1), pl.ds(col0, W)], acc_v)
        for i in range(1, NS):
            pltpu.sync_copy(shared.at[pl.ds(i, 1), pl.ds(col0, W)], tmp_v)

            def mx_body(q, _):
                acc_v[0, pl.ds(q * L, L)] = jnp.maximum(
                    acc_v[0, pl.ds(q * L, L)], tmp_v[0, pl.ds(q * L, L)])
                return 0

            lax.fori_loop(0, W // L, mx_body, 0)
        pltpu.sync_copy(acc_v, m_hbm.at[gi, cid, sid])
        plsc.subcore_barrier()


def _seg_max(dst2, e3):
    f = pl.kernel(
        _seg_max_body,
        out_type=jax.ShapeDtypeStruct((2, NC, NS, 1, W), jnp.float32),
        mesh=_SC_MESH,
        compiler_params=_SC_PARAMS,
        scratch_types=[
            pltpu.VMEM((NCHUNK, CHUNK), jnp.int32),
            pltpu.VMEM((NCHUNK, CHUNK), jnp.float32),
            pltpu.VMEM((NPAD,), jnp.float32),
            pltpu.VMEM_SHARED((NS, NPAD), jnp.float32),
            pltpu.VMEM((1, W), jnp.float32),
            pltpu.VMEM((1, W), jnp.float32),
        ],
    )
    return f(dst2, e3)


def _seg_sum_body(dst_hbm, e_hbm, m4_hbm, den_hbm,
                  d_v, e_v, m_v, t_full, d_tile, shared, acc_v, tmp_v):
    cid = lax.axis_index("c")
    sid = lax.axis_index("s")
    wid = sid * NC + cid
    lane = lax.iota(jnp.int32, L)
    col0 = sid * W
    for gi in range(2):
        pltpu.sync_copy(dst_hbm.at[gi, wid], d_v)
        pltpu.sync_copy(e_hbm.at[gi, wid], e_v)
        pltpu.sync_copy(m4_hbm.at[gi, 0], m_v)
        pltpu.sync_copy(m4_hbm.at[gi, 1], t_full)

        def comb_body(i, _):
            m_v[0, pl.ds(i * L, L)] = jnp.maximum(m_v[0, pl.ds(i * L, L)],
                                                  t_full[0, pl.ds(i * L, L)])
            d_tile[pl.ds(i * L, L)] = jnp.zeros((L,), jnp.float32)
            return 0

        lax.fori_loop(0, NPAD // L, comb_body, 0)

        def chunk_body(ci, _):
            def group_body(g, _):
                d = d_v[ci, pl.ds(g * L, L)]
                e = e_v[ci, pl.ds(g * L, L)]
                md = plsc.load_gather(m_v, [jnp.zeros((L,), jnp.int32), d])
                ex = jnp.exp(e - md)
                ds_, xs_ = lax.sort((d, ex), num_keys=1)
                v = _run_scan(lane, ds_, xs_, lax.add)
                mask = _run_end(lane, ds_)
                plsc.addupdate_scatter(d_tile, [ds_], v, mask=mask)
                return 0

            lax.fori_loop(0, CHUNK // L, group_body, 0)
            return 0

        lax.fori_loop(0, NCHUNK, chunk_body, 0)

        pltpu.sync_copy(d_tile, shared.at[sid])
        plsc.subcore_barrier()
        pltpu.sync_copy(shared.at[pl.ds(0, 1), pl.ds(col0, W)], acc_v)
        for i in range(1, NS):
            pltpu.sync_copy(shared.at[pl.ds(i, 1), pl.ds(col0, W)], tmp_v)

            def ad_body(q, _):
                acc_v[0, pl.ds(q * L, L)] = (acc_v[0, pl.ds(q * L, L)]
                                             + tmp_v[0, pl.ds(q * L, L)])
                return 0

            lax.fori_loop(0, W // L, ad_body, 0)
        pltpu.sync_copy(acc_v, den_hbm.at[gi, cid, sid])
        plsc.subcore_barrier()


def _seg_sum(dst2, e3, m4):
    f = pl.kernel(
        _seg_sum_body,
        out_type=jax.ShapeDtypeStruct((2, NC, NS, 1, W), jnp.float32),
        mesh=_SC_MESH,
        compiler_params=_SC_PARAMS,
        scratch_types=[
            pltpu.VMEM((NCHUNK, CHUNK), jnp.int32),
            pltpu.VMEM((NCHUNK, CHUNK), jnp.float32),
            pltpu.VMEM((1, NPAD), jnp.float32),
            pltpu.VMEM((1, NPAD), jnp.float32),
            pltpu.VMEM((NPAD,), jnp.float32),
            pltpu.VMEM_SHARED((NS, NPAD), jnp.float32),
            pltpu.VMEM((1, W), jnp.float32),
            pltpu.VMEM((1, W), jnp.float32),
        ],
    )
    return f(dst2, e3, m4)


HALF = NPAD // 2


def _seg_out_body(src_hbm, dst_hbm, e_hbm, m4_hbm, d4_hbm, xl0_hbm, xl1_hbm,
                  o_hbm,
                  src_v, dst_v, e_v, m_v, den_v, t_full, rows_v, zer_v,
                  idx_buf, acc_sh, sem):
    cid = lax.axis_index("c")
    sid = lax.axis_index("s")
    wid = sid * NC + cid

    def z_body(i, _):
        for q in range(8):
            zer_v[i, pl.ds(q * L, L)] = jnp.zeros((L,), jnp.float32)
        return 0

    lax.fori_loop(0, 64, z_body, 0)

    for gi, xl_hbm in enumerate((xl0_hbm, xl1_hbm)):
        pltpu.sync_copy(src_hbm.at[gi, wid], src_v)
        pltpu.sync_copy(dst_hbm.at[gi, wid], dst_v)
        pltpu.sync_copy(e_hbm.at[gi, wid], e_v)
        pltpu.sync_copy(m4_hbm.at[gi, 0], m_v)
        pltpu.sync_copy(m4_hbm.at[gi, 1], t_full)

        def combm_body(i, _):
            m_v[0, pl.ds(i * L, L)] = jnp.maximum(m_v[0, pl.ds(i * L, L)],
                                                  t_full[0, pl.ds(i * L, L)])
            return 0

        lax.fori_loop(0, NPAD // L, combm_body, 0)
        pltpu.sync_copy(d4_hbm.at[gi, 0], den_v)
        pltpu.sync_copy(d4_hbm.at[gi, 1], t_full)

        def combd_body(i, _):
            den_v[0, pl.ds(i * L, L)] = (den_v[0, pl.ds(i * L, L)]
                                         + t_full[0, pl.ds(i * L, L)])
            return 0

        lax.fori_loop(0, NPAD // L, combd_body, 0)

        zidx = jnp.zeros((L,), jnp.int32)
        WH = HALF // NS

        for h in range(2):
            h0 = h * HALF
            for q in range(WH // 64):
                pltpu.sync_copy(zer_v, acc_sh.at[pl.ds(sid * WH + q * 64, 64)])
            plsc.subcore_barrier()

            def chunk_body(ci, _):
                pltpu.async_copy(xl_hbm.at[src_v.at[ci]], rows_v, sem).wait()

                def group_body(g, _):
                    d = dst_v[ci, pl.ds(g * L, L)]
                    e = e_v[ci, pl.ds(g * L, L)]
                    md = plsc.load_gather(m_v, [zidx, d])
                    dd = plsc.load_gather(den_v, [zidx, d])
                    dl = d - h0
                    inh = (dl >= 0) & (dl < HALF)
                    idx_buf[0, pl.ds(g * L, L)] = jnp.clip(dl, 0, HALF - 1)
                    alpha = jnp.where(inh, jnp.exp(e - md) / dd, 0.0)
                    for k in range(L):
                        a = alpha[k]
                        for j in range(8):
                            rows_v[g * L + k, pl.ds(j * L, L)] = (
                                rows_v[g * L + k, pl.ds(j * L, L)] * a)
                    return 0

                lax.fori_loop(0, CHUNK // L, group_body, 0)
                pltpu.sync_copy(rows_v, acc_sh.at[idx_buf.at[0]], add=True)
                return 0

            lax.fori_loop(0, NCHUNK, chunk_body, 0)
            plsc.subcore_barrier()
            pltpu.sync_copy(acc_sh.at[pl.ds(sid * WH, WH)],
                            o_hbm.at[gi, cid, pl.ds(h0 + sid * WH, WH)])
            plsc.subcore_barrier()


def _seg_out(src2, dst2, e3, m4, d4, xl0, xl1):
    f = pl.kernel(
        _seg_out_body,
        out_type=jax.ShapeDtypeStruct((2, NC, NPAD, 128), jnp.float32),
        mesh=_SC_MESH,
        compiler_params=_SC_PARAMS,
        scratch_types=[
            pltpu.VMEM((NCHUNK, CHUNK), jnp.int32),
            pltpu.VMEM((NCHUNK, CHUNK), jnp.int32),
            pltpu.VMEM((NCHUNK, CHUNK), jnp.float32),
            pltpu.VMEM((1, NPAD), jnp.float32),
            pltpu.VMEM((1, NPAD), jnp.float32),
            pltpu.VMEM((1, NPAD), jnp.float32),
            pltpu.VMEM((CHUNK, 128), jnp.float32),
            pltpu.VMEM((64, 128), jnp.float32),
            pltpu.VMEM((1, CHUNK), jnp.int32),
            pltpu.VMEM_SHARED((HALF, 128), jnp.float32),
            pltpu.SemaphoreType.DMA,
        ],
    )
    return f(src2, dst2, e3, m4, d4, xl0, xl1)


NPT = NPAD // NW


def _rank_body(score_hbm, batch_hbm, off_hbm, k_hbm, keep_hbm,
               score_v, batch_v, off_v, k_v, keep_v):
    cid = lax.axis_index("c")
    sid = lax.axis_index("s")
    wid = sid * NC + cid
    lane = lax.iota(jnp.int32, L)
    zidx = jnp.zeros((L,), jnp.int32)
    base = wid * NPT
    for gi in range(2):
        pltpu.sync_copy(score_hbm.at[gi], score_v)
        pltpu.sync_copy(batch_hbm.at[gi], batch_v)
        pltpu.sync_copy(off_hbm.at[gi], off_v)
        pltpu.sync_copy(k_hbm.at[gi], k_v)

        def group_body(q, _):
            node0 = base + q * L
            ivec = node0 + lane
            svec = score_v[0, pl.ds(node0, L)]
            bvec = batch_v[0, pl.ds(node0, L)]
            starts = plsc.load_gather(off_v, [zidx, bvec])
            ends = plsc.load_gather(off_v, [zidx, bvec + 1])
            jlo = jnp.min(starts)
            jhi = jnp.max(ends)

            def j_body(j, cnt):
                jv = jnp.full((L,), j, jnp.int32)
                sj = plsc.load_gather(score_v, [zidx, jv])
                bj = plsc.load_gather(batch_v, [zidx, jv])
                hit = (bj == bvec) & ((sj > svec) | ((sj == svec) & (jv < ivec)))
                return cnt + jnp.where(hit, 1, 0)

            cnt = lax.fori_loop(jlo, jhi, j_body, jnp.zeros((L,), jnp.int32))
            kvec = plsc.load_gather(k_v, [zidx, bvec])
            keep_v[0, pl.ds(q * L, L)] = jnp.where(cnt < kvec, 1.0, 0.0)
            return 0

        lax.fori_loop(0, NPT // L, group_body, 0)
        pltpu.sync_copy(keep_v, keep_hbm.at[gi, wid])


def _rank_keep(score2, batch2, off2, k2t):
    f = pl.kernel(
        _rank_body,
        out_type=jax.ShapeDtypeStruct((2, NW, 1, NPT), jnp.float32),
        mesh=_SC_MESH,
        compiler_params=_SC_PARAMS,
        scratch_types=[
            pltpu.VMEM((1, NPAD), jnp.float32),
            pltpu.VMEM((1, NPAD), jnp.int32),
            pltpu.VMEM((1, 32), jnp.int32),
            pltpu.VMEM((1, 32), jnp.int32),
            pltpu.VMEM((1, NPT), jnp.float32),
        ],
    )
    return f(score2, batch2, off2, k2t)


def _edges_padded(edge_index):
    loops = jnp.arange(N, dtype=edge_index.dtype)
    pad = jnp.zeros((EPAD - EP,), edge_index.dtype)
    src = jnp.concatenate([edge_index[0], loops, pad])
    dst = jnp.concatenate([edge_index[1], loops, pad])
    return (src.reshape(NW, NCHUNK, CHUNK), dst.reshape(NW, NCHUNK, CHUNK))


def _global_pools_dense(x, keepf, onehot, counts_kept):
    sums = onehot.T @ (x * keepf[:, None])
    gmean = sums / jnp.maximum(counts_kept, 1.0)[:, None]
    maxes = []
    keepb = keepf > 0.5
    for g in range(G):
        mask = (onehot[:, g] > 0.5) & keepb
        mg = jnp.max(jnp.where(mask[:, None], x, NEG), axis=0)
        maxes.append(mg)
    gmax = jnp.stack(maxes)
    gmax = jnp.where(gmax <= NEG * 0.5, 0.0, gmax)
    return jnp.concatenate([gmax, gmean], axis=1)


def kernel(source_x, source_edge_index, source_batch, target_x, target_edge_index, target_batch, Wl1, bl1, Wr1, br1, att1, bias1, Wl2, bl2, Wr2, br2, att2, bias2, pool_w1, pool_w2, W1, b1, W2, b2, W3, b3):
    srcA, dstA = _edges_padded(source_edge_index)
    srcB, dstB = _edges_padded(target_edge_index)
    src2 = jnp.stack([srcA, srcB])
    dst2 = jnp.stack([dstA, dstB])
    att2g = jnp.stack([att1, att2])

    xl0, xr0 = _linear2(source_x, Wl1, bl1, Wr1, br1)
    xl1, xr1 = _linear2(target_x, Wl2, bl2, Wr2, br2)

    e3 = _edge_scores(xl0, xr0, xl1, xr1, src2, dst2, att2g)
    m4 = _seg_max(dst2, e3).reshape(2, NC, 1, NPAD)
    d4 = _seg_sum(dst2, e3, m4).reshape(2, NC, 1, NPAD)
    opart = _seg_out(src2, dst2, e3, m4, d4, xl0, xl1)

    s = jax.nn.leaky_relu(opart[0, 0, :N] + opart[0, 1, :N] + bias1,
                          negative_slope=0.01)
    t = jax.nn.leaky_relu(opart[1, 0, :N] + opart[1, 1, :N] + bias2,
                          negative_slope=0.01)

    giota = jnp.arange(G, dtype=jnp.int32)
    feats = []
    scaled_keep = []
    for x_, batch, pw in ((s, source_batch, pool_w1), (t, target_batch, pool_w2)):
        score = jnp.tanh((x_ @ pw) / jnp.linalg.norm(pw))
        onehot = (batch[:, None] == giota[None, :]).astype(jnp.float32)
        counts = jnp.sum(onehot, axis=0)
        ends = jnp.cumsum(counts).astype(jnp.int32)
        off = jnp.concatenate([jnp.zeros((1,), jnp.int32), ends,
                               jnp.full((32 - G - 1,), N, jnp.int32)])
        kv = jnp.ceil(RATIO * counts).astype(jnp.int32)
        kv = jnp.concatenate([kv, jnp.zeros((32 - G,), jnp.int32)])
        scaled_keep.append((x_ * score[:, None], batch, onehot, counts,
                            score, off, kv))

    score2 = jnp.stack([
        jnp.pad(scaled_keep[0][4], (0, NPAD - N), constant_values=-2.0),
        jnp.pad(scaled_keep[1][4], (0, NPAD - N), constant_values=-2.0),
    ]).reshape(2, 1, NPAD)
    batch2 = jnp.stack([
        jnp.pad(source_batch, (0, NPAD - N), constant_values=G),
        jnp.pad(target_batch, (0, NPAD - N), constant_values=G),
    ]).reshape(2, 1, NPAD)
    off2 = jnp.stack([scaled_keep[0][5], scaled_keep[1][5]]).reshape(2, 1, 32)
    k2t = jnp.stack([scaled_keep[0][6], scaled_keep[1][6]]).reshape(2, 1, 32)
    keep2 = _rank_keep(score2, batch2, off2, k2t).reshape(2, NPAD)[:, :N]

    for gi, (xsc, batch, onehot, counts, _, _, kv) in enumerate(scaled_keep):
        keepf = keep2[gi]
        counts_kept = onehot.T @ keepf
        feats.append(_global_pools_dense(xsc, keepf, onehot, counts_kept))

    h = jnp.concatenate([feats[0], feats[1]], axis=1)
    h = jax.nn.leaky_relu(h @ W1 + b1, negative_slope=0.01)
    h = jax.nn.leaky_relu(h @ W2 + b2, negative_slope=0.01)
    h = h @ W3 + b3
    return jax.nn.log_softmax(h, axis=-1)
